# baseline (speedup 1.0000x reference)
import functools

import jax
import jax.numpy as jnp
from jax import lax
from jax.experimental import pallas as pl
from jax.experimental.pallas import tpu as pltpu
from jax.experimental.pallas import tpu_sc as plsc

_NC = 2
_NS = 16
_NW = _NC * _NS
_CHUNK = 128
_PAIR = 1
_BLKCH = 16
_LANES = 16


def _rup(v, m):
    return -(-v // m) * m


@functools.lru_cache(maxsize=None)
def _make_segsum(d, wch, acc_rows, slabz, wb):
    npair = wch // _PAIR
    ngrp = wch // _BLKCH
    mesh = plsc.VectorSubcoreMesh(core_axis_name="c", subcore_axis_name="s")

    @functools.partial(
        pl.kernel,
        out_type=jax.ShapeDtypeStruct((_NC, wb * _NS, d), jnp.float32),
        mesh=mesh,
        scratch_types=[
            pltpu.VMEM((2, _BLKCH, _CHUNK), jnp.int32),
            pltpu.VMEM((2, _BLKCH, _CHUNK), jnp.int32),
            pltpu.VMEM((_PAIR * _CHUNK, d), jnp.float32),
            pltpu.VMEM((_PAIR * _CHUNK, d), jnp.float32),
            pltpu.VMEM_SHARED((acc_rows, d), jnp.float32),
            pltpu.SemaphoreType.DMA,
            pltpu.SemaphoreType.DMA,
            pltpu.SemaphoreType.DMA,
            pltpu.SemaphoreType.DMA,
            pltpu.SemaphoreType.DMA,
        ],
        compiler_params=pltpu.CompilerParams(needs_layout_passes=False),
    )
    def segsum(sup_hbm, src_hbm, dst_hbm, zero_hbm, out_hbm,
               sidx, didx, rows_a, rows_b, acc, sem_a, sem_b,
               ssem_a, ssem_b, isem):
        c = lax.axis_index("c")
        s = lax.axis_index("s")
        w = s * _NC + c
        pltpu.sync_copy(zero_hbm.at[s], acc.at[pl.ds(s * slabz, slabz)])
        pltpu.sync_copy(src_hbm.at[w].at[pl.ds(0, _BLKCH)], sidx.at[0])
        pltpu.sync_copy(dst_hbm.at[w].at[pl.ds(0, _BLKCH)], didx.at[0])
        if ngrp > 1:
            pltpu.async_copy(src_hbm.at[w].at[pl.ds(_BLKCH, _BLKCH)],
                             sidx.at[1], isem)
            pltpu.async_copy(dst_hbm.at[w].at[pl.ds(_BLKCH, _BLKCH)],
                             didx.at[1], isem)
        plsc.subcore_barrier()

        rows = (rows_a, rows_b)
        sems = (sem_a, sem_b)
        ssems = (ssem_a, ssem_b)
        ppg = _BLKCH // _PAIR

        def gather(g, kp, buf, sem):
            pltpu.async_copy(sup_hbm.at[sidx.at[g % 2].at[kp]], buf, sem)

        def wait_gather(buf, sem):
            pltpu.make_async_copy(sup_hbm.at[sidx.at[0].at[0]],
                                  buf, sem).wait()

        def wait_iload(g):
            pltpu.make_async_copy(src_hbm.at[w].at[pl.ds(0, _BLKCH)],
                                  sidx.at[g % 2], isem).wait()
            pltpu.make_async_copy(dst_hbm.at[w].at[pl.ds(0, _BLKCH)],
                                  didx.at[g % 2], isem).wait()

        def scatter(g, kp, b):
            pltpu.async_copy(rows[b], acc.at[didx.at[g % 2].at[kp]],
                             ssems[b], add=True)

        def wait_scatter(b):
            pltpu.make_async_copy(rows[b], acc.at[didx.at[0].at[0]],
                                  ssems[b]).wait()

        gather(0, 0, rows_a, sem_a)
        for g in range(ngrp):
            if 1 <= g < ngrp - 1:
                pltpu.async_copy(
                    src_hbm.at[w].at[pl.ds((g + 1) * _BLKCH, _BLKCH)],
                    sidx.at[(g + 1) % 2], isem)
                pltpu.async_copy(
                    dst_hbm.at[w].at[pl.ds((g + 1) * _BLKCH, _BLKCH)],
                    didx.at[(g + 1) % 2], isem)
            for kp in range(ppg):
                p = g * ppg + kp
                b = p % 2
                wait_gather(rows[b], sems[b])
                scatter(g, kp, b)
                if p + 1 < npair:
                    if p >= 1:
                        wait_scatter(1 - b)
                    if kp == ppg - 1:
                        wait_iload(g + 1)
                        gather(g + 1, 0, rows[1 - b], sems[1 - b])
                    else:
                        gather(g, kp + 1, rows[1 - b], sems[1 - b])

        wait_scatter(0)
        wait_scatter(1)
        plsc.subcore_barrier()
        pltpu.sync_copy(acc.at[pl.ds(s * wb, wb)],
                        out_hbm.at[c].at[pl.ds(s * wb, wb)])

    return segsum


_BLK = 1000


def _mm_bias_body(x_ref, w_ref, b_ref, o_ref):
    o_ref[...] = (jnp.dot(x_ref[...], w_ref[...],
                          preferred_element_type=jnp.float32) + b_ref[...])


def _mm_bias(x2, w, b):
    n, k = x2.shape
    m = w.shape[1]
    return pl.pallas_call(
        _mm_bias_body,
        grid=(n // _BLK,),
        in_specs=[
            pl.BlockSpec((_BLK, k), lambda i: (i, 0)),
            pl.BlockSpec((k, m), lambda i: (0, 0)),
            pl.BlockSpec((1, m), lambda i: (0, 0)),
        ],
        out_specs=pl.BlockSpec((_BLK, m), lambda i: (i, 0)),
        out_shape=jax.ShapeDtypeStruct((n, m), jnp.float32),
    )(x2, w, b.reshape(1, m))


def _combine_gru_body(flags_ref, aggA_ref, aggB_ref, res0_ref, resc_ref,
                      sup0n_ref, w2_ref, b2_ref, wn_ref, bn_ref,
                      h_ref, wh_ref, bh_ref,
                      out_ref, supn_ref, h_out_ref):
    d = resc_ref.shape[1]
    is_l0 = flags_ref[0, 0] > 0.5
    is_last = flags_ref[0, 1] > 0.5
    next_l0 = flags_ref[0, 2] > 0.5
    res = jnp.where(is_l0, res0_ref[...], resc_ref[...])
    agg = aggA_ref[...] + aggB_ref[...]
    r = (jnp.dot(res, w2_ref[...],
                 preferred_element_type=jnp.float32) + b2_ref[...])
    g = jax.nn.sigmoid(agg + r)
    o = g * jnp.tanh(agg) + (1.0 - g) * r
    out_ref[...] = o
    gx = (jnp.dot(o, wn_ref[...],
                  preferred_element_type=jnp.float32) + bn_ref[...])
    supn_ref[...] = jnp.where(next_l0, sup0n_ref[...], gx[:, :d])
    h = h_ref[...]
    gh = (jnp.dot(h, wh_ref[...],
                  preferred_element_type=jnp.float32) + bh_ref[...])
    i_r, i_i, i_n = gx[:, :d], gx[:, d:2 * d], gx[:, 2 * d:]
    h_r, h_i, h_n = gh[:, :d], gh[:, d:2 * d], gh[:, 2 * d:]
    rg = jax.nn.sigmoid(i_r + h_r)
    ig = jax.nn.sigmoid(i_i + h_i)
    ng = jnp.tanh(i_n + rg * h_n)
    h_new = ng + ig * (h - ng)
    h_out_ref[...] = jnp.where(is_last, h_new, h)


def _combine_gru(flags, aggA, aggB, res0, resc, sup0n, w2, b2, wn, bn,
                 h, wh, bh):
    n, d = resc.shape
    m = wn.shape[1]
    return pl.pallas_call(
        _combine_gru_body,
        grid=(n // _BLK,),
        in_specs=[
            pl.BlockSpec((1, 128), lambda i: (0, 0)),
            pl.BlockSpec((_BLK, d), lambda i: (i, 0)),
            pl.BlockSpec((_BLK, d), lambda i: (i, 0)),
            pl.BlockSpec((_BLK, d), lambda i: (i, 0)),
            pl.BlockSpec((_BLK, d), lambda i: (i, 0)),
            pl.BlockSpec((_BLK, d), lambda i: (i, 0)),
            pl.BlockSpec((d, d), lambda i: (0, 0)),
            pl.BlockSpec((1, d), lambda i: (0, 0)),
            pl.BlockSpec((d, m), lambda i: (0, 0)),
            pl.BlockSpec((1, m), lambda i: (0, 0)),
            pl.BlockSpec((_BLK, d), lambda i: (i, 0)),
            pl.BlockSpec((d, m), lambda i: (0, 0)),
            pl.BlockSpec((1, m), lambda i: (0, 0)),
        ],
        out_specs=[
            pl.BlockSpec((_BLK, d), lambda i: (i, 0)),
            pl.BlockSpec((_BLK, d), lambda i: (i, 0)),
            pl.BlockSpec((_BLK, d), lambda i: (i, 0)),
        ],
        out_shape=[
            jax.ShapeDtypeStruct((n, d), jnp.float32),
            jax.ShapeDtypeStruct((n, d), jnp.float32),
            jax.ShapeDtypeStruct((n, d), jnp.float32),
        ],
    )(flags, aggA, aggB, res0, resc, sup0n, w2, b2.reshape(1, d), wn,
      bn.reshape(1, m), h, wh, bh.reshape(1, m))


def kernel(x, edge_index, W1, b1, W2, b2, Wx, bx, Wh, bh):
    t_steps, n_nodes, d = x.shape
    layers = W1.shape[1]
    n_edges = edge_index.shape[1]

    wb = _rup(-(-n_nodes // _NS), 8)
    n_pad = wb * _NS
    slabz = _rup(-(-(n_pad + 8) // _NS), 8)
    acc_rows = slabz * _NS

    per_worker_unit = _CHUNK * _NW
    epad = _rup(n_edges, per_worker_unit * _BLKCH)
    wch = epad // per_worker_unit

    pad = epad - n_edges
    pad_i = jnp.arange(pad, dtype=jnp.int32)
    srcp = jnp.concatenate(
        [edge_index[0], pad_i % n_nodes]).reshape(_NW, wch, _CHUNK)
    dstp = jnp.concatenate(
        [edge_index[1], n_pad + pad_i % (acc_rows - n_pad)]
    ).reshape(_NW, wch, _CHUNK)
    zero_blk = jnp.zeros((_NS, slabz, d), jnp.float32)

    segsum = _make_segsum(d, wch, acc_rows, slabz, wb)

    sup0 = jnp.stack([_mm_bias(x[t], W1[t, 0], b1[t, 0])
                      for t in range(t_steps)])

    w2s, b2s, wns, bns, sup0n, res0s, flgs = [], [], [], [], [], [], []
    znd = jnp.zeros((n_nodes, d), jnp.float32)
    steps = t_steps * layers
    for t in range(t_steps):
        for l in range(layers):
            i = t * layers + l
            w2s.append(W2[t, l])
            b2s.append(b2[t, l])
            if l + 1 < layers:
                wns.append(jnp.pad(W1[t, l + 1], ((0, 0), (0, 2 * d))))
                bns.append(jnp.pad(b1[t, l + 1], (0, 2 * d)))
            else:
                wns.append(Wx)
                bns.append(bx)
            next_l0 = (i + 1 < steps) and (l + 1 == layers)
            sup0n.append(sup0[t + 1] if next_l0 else znd)
            res0s.append(x[t] if l == 0 else znd)
            fl = jnp.zeros((1, 128), jnp.float32)
            fl = fl.at[0, 0].set(1.0 if l == 0 else 0.0)
            fl = fl.at[0, 1].set(1.0 if l == layers - 1 else 0.0)
            fl = fl.at[0, 2].set(1.0 if next_l0 else 0.0)
            flgs.append(fl)
    xs = (jnp.stack(w2s), jnp.stack(b2s), jnp.stack(wns), jnp.stack(bns),
          jnp.stack(sup0n), jnp.stack(res0s), jnp.stack(flgs))

    def step(carry, xso):
        res, sup, h = carry
        w2i, b2i, wni, bni, sup0ni, res0i, fli = xso
        parts = segsum(sup, srcp, dstp, zero_blk)
        res_out, sup_out, h_out = _combine_gru(
            fli, parts[0], parts[1], res0i, res, sup0ni, w2i, b2i, wni, bni,
            h, Wh, bh)
        return (res_out, sup_out, h_out), h_out

    init = (znd, sup0[0], znd)
    _, hs = lax.scan(step, init, xs)
    return hs[layers - 1::layers]


# speedup vs baseline: 1.0284x; 1.0284x over previous
import functools

import jax
import jax.numpy as jnp
from jax import lax
from jax.experimental import pallas as pl
from jax.experimental.pallas import tpu as pltpu
from jax.experimental.pallas import tpu_sc as plsc

_NC = 2
_NS = 16
_NW = _NC * _NS
_CHUNK = 128
_PAIR = 1
_BLKCH = 16
_LANES = 16


def _rup(v, m):
    return -(-v // m) * m


@functools.lru_cache(maxsize=None)
def _make_segsum(d, wch, acc_rows, slabz, wb):
    npair = wch // _PAIR
    ngrp = wch // _BLKCH
    mesh = plsc.VectorSubcoreMesh(core_axis_name="c", subcore_axis_name="s")

    @functools.partial(
        pl.kernel,
        out_type=jax.ShapeDtypeStruct((_NC, wb * _NS, d), jnp.float32),
        mesh=mesh,
        scratch_types=[
            pltpu.VMEM((2, _BLKCH, _CHUNK), jnp.int32),
            pltpu.VMEM((2, _BLKCH, _CHUNK), jnp.int32),
            pltpu.VMEM((_PAIR * _CHUNK, d), jnp.float32),
            pltpu.VMEM((_PAIR * _CHUNK, d), jnp.float32),
            pltpu.VMEM_SHARED((acc_rows, d), jnp.float32),
            pltpu.SemaphoreType.DMA,
            pltpu.SemaphoreType.DMA,
            pltpu.SemaphoreType.DMA,
            pltpu.SemaphoreType.DMA,
            pltpu.SemaphoreType.DMA,
        ],
        compiler_params=pltpu.CompilerParams(needs_layout_passes=False),
    )
    def segsum(sup_hbm, src_hbm, dst_hbm, zero_hbm, out_hbm,
               sidx, didx, rows_a, rows_b, acc, sem_a, sem_b,
               ssem_a, ssem_b, isem):
        c = lax.axis_index("c")
        s = lax.axis_index("s")
        w = s * _NC + c
        pltpu.sync_copy(zero_hbm, acc.at[pl.ds(s * slabz, slabz)])
        pltpu.sync_copy(src_hbm.at[w].at[pl.ds(0, _BLKCH)], sidx.at[0])
        pltpu.sync_copy(dst_hbm.at[w].at[pl.ds(0, _BLKCH)], didx.at[0])
        if ngrp > 1:
            pltpu.async_copy(src_hbm.at[w].at[pl.ds(_BLKCH, _BLKCH)],
                             sidx.at[1], isem)
            pltpu.async_copy(dst_hbm.at[w].at[pl.ds(_BLKCH, _BLKCH)],
                             didx.at[1], isem)
        plsc.subcore_barrier()

        rows = (rows_a, rows_b)
        sems = (sem_a, sem_b)
        ssems = (ssem_a, ssem_b)
        ppg = _BLKCH // _PAIR

        def gather(g, kp, buf, sem):
            pltpu.async_copy(sup_hbm.at[sidx.at[g % 2].at[kp]], buf, sem)

        def wait_gather(buf, sem):
            pltpu.make_async_copy(sup_hbm.at[sidx.at[0].at[0]],
                                  buf, sem).wait()

        def wait_iload(g):
            pltpu.make_async_copy(src_hbm.at[w].at[pl.ds(0, _BLKCH)],
                                  sidx.at[g % 2], isem).wait()
            pltpu.make_async_copy(dst_hbm.at[w].at[pl.ds(0, _BLKCH)],
                                  didx.at[g % 2], isem).wait()

        def scatter(g, kp, b):
            pltpu.async_copy(rows[b], acc.at[didx.at[g % 2].at[kp]],
                             ssems[b], add=True)

        def wait_scatter(b):
            pltpu.make_async_copy(rows[b], acc.at[didx.at[0].at[0]],
                                  ssems[b]).wait()

        gather(0, 0, rows_a, sem_a)
        for g in range(ngrp):
            if 1 <= g < ngrp - 1:
                pltpu.async_copy(
                    src_hbm.at[w].at[pl.ds((g + 1) * _BLKCH, _BLKCH)],
                    sidx.at[(g + 1) % 2], isem)
                pltpu.async_copy(
                    dst_hbm.at[w].at[pl.ds((g + 1) * _BLKCH, _BLKCH)],
                    didx.at[(g + 1) % 2], isem)
            for kp in range(ppg):
                p = g * ppg + kp
                b = p % 2
                wait_gather(rows[b], sems[b])
                scatter(g, kp, b)
                if p + 1 < npair:
                    if p >= 1:
                        wait_scatter(1 - b)
                    if kp == ppg - 1:
                        wait_iload(g + 1)
                        gather(g + 1, 0, rows[1 - b], sems[1 - b])
                    else:
                        gather(g, kp + 1, rows[1 - b], sems[1 - b])

        wait_scatter(0)
        wait_scatter(1)
        plsc.subcore_barrier()
        pltpu.sync_copy(acc.at[pl.ds(s * wb, wb)],
                        out_hbm.at[c].at[pl.ds(s * wb, wb)])

    return segsum


_BLK = 1000


def _mm_bias_body(x_ref, w_ref, b_ref, o_ref):
    o_ref[...] = (jnp.dot(x_ref[...], w_ref[...],
                          preferred_element_type=jnp.float32) + b_ref[...])


def _mm_bias(x2, w, b):
    n, k = x2.shape
    m = w.shape[1]
    return pl.pallas_call(
        _mm_bias_body,
        grid=(n // _BLK,),
        in_specs=[
            pl.BlockSpec((_BLK, k), lambda i: (i, 0)),
            pl.BlockSpec((k, m), lambda i: (0, 0)),
            pl.BlockSpec((1, m), lambda i: (0, 0)),
        ],
        out_specs=pl.BlockSpec((_BLK, m), lambda i: (i, 0)),
        out_shape=jax.ShapeDtypeStruct((n, m), jnp.float32),
    )(x2, w, b.reshape(1, m))


def _combine_gru_body(flags_ref, aggA_ref, aggB_ref, res0_ref, resc_ref,
                      w2_ref, b2_ref, wn_ref, bn_ref, h_ref, wh_ref, bh_ref,
                      out_ref, supn_ref, h_out_ref):
    d = resc_ref.shape[1]
    is_l0 = flags_ref[0, 0] > 0.5
    is_last = flags_ref[0, 1] > 0.5
    res = jnp.where(is_l0, res0_ref[...], resc_ref[...])
    agg = aggA_ref[...] + aggB_ref[...]
    r = (jnp.dot(res, w2_ref[...],
                 preferred_element_type=jnp.float32) + b2_ref[...])
    g = jax.nn.sigmoid(agg + r)
    o = g * jnp.tanh(agg) + (1.0 - g) * r
    out_ref[...] = o
    gx = (jnp.dot(o, wn_ref[...],
                  preferred_element_type=jnp.float32) + bn_ref[...])
    supn_ref[...] = gx[:, :d]
    h = h_ref[...]
    gh = (jnp.dot(h, wh_ref[...],
                  preferred_element_type=jnp.float32) + bh_ref[...])
    i_r, i_i, i_n = gx[:, :d], gx[:, d:2 * d], gx[:, 2 * d:]
    h_r, h_i, h_n = gh[:, :d], gh[:, d:2 * d], gh[:, 2 * d:]
    rg = jax.nn.sigmoid(i_r + h_r)
    ig = jax.nn.sigmoid(i_i + h_i)
    ng = jnp.tanh(i_n + rg * h_n)
    h_new = ng + ig * (h - ng)
    h_out_ref[...] = jnp.where(is_last, h_new, h)


def _combine_gru(flags, aggA, aggB, res0, resc, w2, b2, wn, bn, h, wh, bh):
    n, d = resc.shape
    m = wn.shape[1]
    return pl.pallas_call(
        _combine_gru_body,
        grid=(n // _BLK,),
        in_specs=[
            pl.BlockSpec((1, 128), lambda i: (0, 0)),
            pl.BlockSpec((_BLK, d), lambda i: (i, 0)),
            pl.BlockSpec((_BLK, d), lambda i: (i, 0)),
            pl.BlockSpec((_BLK, d), lambda i: (i, 0)),
            pl.BlockSpec((_BLK, d), lambda i: (i, 0)),
            pl.BlockSpec((d, d), lambda i: (0, 0)),
            pl.BlockSpec((1, d), lambda i: (0, 0)),
            pl.BlockSpec((d, m), lambda i: (0, 0)),
            pl.BlockSpec((1, m), lambda i: (0, 0)),
            pl.BlockSpec((_BLK, d), lambda i: (i, 0)),
            pl.BlockSpec((d, m), lambda i: (0, 0)),
            pl.BlockSpec((1, m), lambda i: (0, 0)),
        ],
        out_specs=[
            pl.BlockSpec((_BLK, d), lambda i: (i, 0)),
            pl.BlockSpec((_BLK, d), lambda i: (i, 0)),
            pl.BlockSpec((_BLK, d), lambda i: (i, 0)),
        ],
        out_shape=[
            jax.ShapeDtypeStruct((n, d), jnp.float32),
            jax.ShapeDtypeStruct((n, d), jnp.float32),
            jax.ShapeDtypeStruct((n, d), jnp.float32),
        ],
    )(flags, aggA, aggB, res0, resc, w2, b2.reshape(1, d), wn,
      bn.reshape(1, m), h, wh, bh.reshape(1, m))


def kernel(x, edge_index, W1, b1, W2, b2, Wx, bx, Wh, bh):
    t_steps, n_nodes, d = x.shape
    layers = W1.shape[1]
    n_edges = edge_index.shape[1]

    wb = _rup(-(-n_nodes // _NS), 8)
    n_pad = wb * _NS
    slabz = _rup(-(-(n_pad + 8) // _NS), 8)
    acc_rows = slabz * _NS

    per_worker_unit = _CHUNK * _NW
    epad = _rup(n_edges, per_worker_unit * _BLKCH)
    wch = epad // per_worker_unit

    pad = epad - n_edges
    pad_i = jnp.arange(pad, dtype=jnp.int32)
    srcp = jnp.concatenate(
        [edge_index[0], pad_i % n_nodes]).reshape(_NW, wch, _CHUNK)
    dstp = jnp.concatenate(
        [edge_index[1], n_pad + pad_i % (acc_rows - n_pad)]
    ).reshape(_NW, wch, _CHUNK)
    zero_blk = jnp.zeros((slabz, d), jnp.float32)

    segsum = _make_segsum(d, wch, acc_rows, slabz, wb)

    sup0 = jnp.stack([_mm_bias(x[t], W1[t, 0], b1[t, 0])
                      for t in range(t_steps)])

    w2s, b2s, wns, bns, sup0s, res0s, flgs = [], [], [], [], [], [], []
    znd = jnp.zeros((n_nodes, d), jnp.float32)
    for t in range(t_steps):
        for l in range(layers):
            w2s.append(W2[t, l])
            b2s.append(b2[t, l])
            if l + 1 < layers:
                wns.append(jnp.pad(W1[t, l + 1], ((0, 0), (0, 2 * d))))
                bns.append(jnp.pad(b1[t, l + 1], (0, 2 * d)))
            else:
                wns.append(Wx)
                bns.append(bx)
            sup0s.append(sup0[t] if l == 0 else znd)
            res0s.append(x[t] if l == 0 else znd)
            fl = jnp.zeros((1, 128), jnp.float32)
            fl = fl.at[0, 0].set(1.0 if l == 0 else 0.0)
            fl = fl.at[0, 1].set(1.0 if l == layers - 1 else 0.0)
            flgs.append(fl)
    xs = (jnp.stack(w2s), jnp.stack(b2s), jnp.stack(wns), jnp.stack(bns),
          jnp.stack(sup0s), jnp.stack(res0s), jnp.stack(flgs))

    def step(carry, xso):
        res, sup, h = carry
        w2i, b2i, wni, bni, sup0i, res0i, fli = xso
        is_l0 = fli[0, 0] > 0.5
        sup_in = jnp.where(is_l0, sup0i, sup)
        parts = segsum(sup_in, srcp, dstp, zero_blk)
        res_out, sup_out, h_out = _combine_gru(
            fli, parts[0], parts[1], res0i, res, w2i, b2i, wni, bni,
            h, Wh, bh)
        return (res_out, sup_out, h_out), h_out

    init = (znd, znd, znd)
    _, hs = lax.scan(step, init, xs)
    return hs[layers - 1::layers]
